# Initial kernel scaffold; baseline (speedup 1.0000x reference)
import functools

import jax
import jax.numpy as jnp
from jax import lax
from jax.experimental import pallas as pl
from jax.experimental.pallas import tpu as pltpu
from jax.experimental.pallas import tpu_sc as plsc

N = 10000
E = 320000
C = 128
NEG = 0.2

TILES = 32
B = 128
NB = 79
EPT = NB * B
PAD = TILES * EPT - E
ROW_W = C + 16
ACC_ROWS = 10240
BLK = 1000


def _prologue_body(x_ref, w_ref, asrc_ref, adst_ref,
                   h_ref, as_ref, ad_ref, c_ref, m_ref):
    i = pl.program_id(0)
    h = jnp.dot(x_ref[...], w_ref[...], preferred_element_type=jnp.float32)
    h_ref[...] = h
    a_s = jnp.sum(h * asrc_ref[...], axis=1, keepdims=True)
    a_d = jnp.sum(h * adst_ref[...], axis=1, keepdims=True)
    as_ref[...] = a_s
    ad_ref[...] = a_d
    bs = jnp.max(a_s)
    bd = jnp.max(a_d)

    @pl.when(i == 0)
    def _():
        m_ref[0] = bs
        m_ref[1] = bd

    @pl.when(i > 0)
    def _():
        m_ref[0] = jnp.maximum(m_ref[0], bs)
        m_ref[1] = jnp.maximum(m_ref[1], bd)

    @pl.when(i == N // BLK - 1)
    def _():
        s = m_ref[0] + m_ref[1]
        c_ref[0, 0] = jnp.maximum(s, NEG * s)


_prologue = pl.pallas_call(
    _prologue_body,
    grid=(N // BLK,),
    in_specs=[
        pl.BlockSpec((BLK, C), lambda i: (i, 0)),
        pl.BlockSpec((C, C), lambda i: (0, 0)),
        pl.BlockSpec((1, C), lambda i: (0, 0)),
        pl.BlockSpec((1, C), lambda i: (0, 0)),
    ],
    out_specs=[
        pl.BlockSpec((BLK, C), lambda i: (i, 0)),
        pl.BlockSpec((BLK, 1), lambda i: (i, 0)),
        pl.BlockSpec((BLK, 1), lambda i: (i, 0)),
        pl.BlockSpec((1, 1), lambda i: (0, 0)),
    ],
    out_shape=[
        jax.ShapeDtypeStruct((N, C), jnp.float32),
        jax.ShapeDtypeStruct((N, 1), jnp.float32),
        jax.ShapeDtypeStruct((N, 1), jnp.float32),
        jax.ShapeDtypeStruct((1, 1), jnp.float32),
    ],
    scratch_shapes=[pltpu.SMEM((2,), jnp.float32)],
)


_sc_mesh = plsc.VectorSubcoreMesh(
    core_axis_name="c", subcore_axis_name="s", num_cores=2, num_subcores=16)


@functools.partial(
    pl.kernel,
    out_type=jax.ShapeDtypeStruct((2 * N, ROW_W), jnp.float32),
    mesh=_sc_mesh,
    scratch_types=[
        pltpu.VMEM((N,), jnp.float32),
        pltpu.VMEM((N,), jnp.float32),
        pltpu.VMEM((16,), jnp.float32),
        pltpu.VMEM((NB, B), jnp.int32),
        pltpu.VMEM((NB, B), jnp.int32),
        pltpu.VMEM((B, C), jnp.float32),
        pltpu.VMEM((B, ROW_W), jnp.float32),
        pltpu.VMEM((B,), jnp.float32),
        pltpu.VMEM_SHARED((ACC_ROWS, ROW_W), jnp.float32),
        pltpu.SemaphoreType.DMA,
    ],
)
def _edge_pass(h_hbm, as_hbm, ad_hbm, c_hbm, src_hbm, dst_hbm, out_hbm,
               as_v, ad_v, c_v, srcv, dstv, rows_v, msg_v, wbuf, acc, sem):
    core = lax.axis_index("c")
    sub = lax.axis_index("s")
    tile = sub * 2 + core

    pltpu.sync_copy(as_hbm, as_v)
    pltpu.sync_copy(ad_hbm, ad_v)
    pltpu.sync_copy(c_hbm, c_v)
    pltpu.sync_copy(src_hbm.at[tile], srcv)
    pltpu.sync_copy(dst_hbm.at[tile], dstv)

    zero16 = jnp.zeros((16,), jnp.float32)

    @pl.loop(0, B)
    def _(i):
        @pl.loop(0, ROW_W // 16)
        def _(j):
            msg_v[i, pl.ds(j * 16, 16)] = zero16

    @pl.loop(0, ACC_ROWS // (16 * B))
    def _(k):
        pltpu.sync_copy(msg_v, acc.at[pl.ds(sub * 640 + k * B, B)])

    plsc.subcore_barrier()

    iota16 = lax.iota(jnp.int32, 16)
    onehot0 = jnp.where(iota16 == 0, 1.0, 0.0).astype(jnp.float32)
    cvec = c_v[...]

    @pl.loop(0, NB)
    def _(b):
        gcp = pltpu.async_copy(h_hbm.at[srcv.at[b]], rows_v, sem)

        @pl.loop(0, B // 16)
        def _(g):
            sidx = srcv[b, pl.ds(g * 16, 16)]
            didx = dstv[b, pl.ds(g * 16, 16)]
            e = plsc.load_gather(as_v, [sidx]) + plsc.load_gather(ad_v, [didx])
            e = jnp.maximum(e, NEG * e)
            wbuf[pl.ds(g * 16, 16)] = jnp.exp(e - cvec)

        gcp.wait()

        @pl.loop(0, B)
        def _(i):
            ws = wbuf[i]
            for j in range(C // 16):
                msg_v[i, pl.ds(j * 16, 16)] = rows_v[i, pl.ds(j * 16, 16)] * ws
            msg_v[i, pl.ds(C, 16)] = onehot0 * ws

        pltpu.sync_copy(msg_v, acc.at[dstv.at[b]], add=True)

    plsc.subcore_barrier()

    pltpu.sync_copy(acc.at[pl.ds(sub * 625, 625)],
                    out_hbm.at[pl.ds(core * N + sub * 625, 625)])


def _combine_body(p_ref, h_ref, as_ref, ad_ref, c_ref, b_ref, o_ref):
    es = as_ref[...] + ad_ref[...]
    es = jnp.maximum(es, NEG * es)
    ws = jnp.exp(es - c_ref[0, 0])
    p = p_ref[...]
    num = p[0, :, :C] + p[1, :, :C] + ws * h_ref[...]
    den = p[0, :, C:C + 1] + p[1, :, C:C + 1] + ws
    o_ref[...] = num / (den + 1e-16) + b_ref[...]


_combine = pl.pallas_call(
    _combine_body,
    grid=(N // BLK,),
    in_specs=[
        pl.BlockSpec((2, BLK, ROW_W), lambda i: (0, i, 0)),
        pl.BlockSpec((BLK, C), lambda i: (i, 0)),
        pl.BlockSpec((BLK, 1), lambda i: (i, 0)),
        pl.BlockSpec((BLK, 1), lambda i: (i, 0)),
        pl.BlockSpec((1, 1), lambda i: (0, 0)),
        pl.BlockSpec((1, C), lambda i: (0, 0)),
    ],
    out_specs=pl.BlockSpec((BLK, C), lambda i: (i, 0)),
    out_shape=jax.ShapeDtypeStruct((N, C), jnp.float32),
)


def kernel(x, edge_index, W, att_src, att_dst, bias):
    src = edge_index[0].astype(jnp.int32)
    dst = edge_index[1].astype(jnp.int32)
    src_p = jnp.concatenate(
        [src, jnp.zeros((PAD,), jnp.int32)]).reshape(TILES, NB, B)
    dst_p = jnp.concatenate(
        [dst, jnp.full((PAD,), N, jnp.int32)]).reshape(TILES, NB, B)

    h, a_s, a_d, c = _prologue(
        x, W, att_src.reshape(1, C), att_dst.reshape(1, C))
    cvec = jnp.broadcast_to(c[0], (16,))

    parts = _edge_pass(h, a_s.reshape(N), a_d.reshape(N), cvec, src_p, dst_p)
    parts = parts.reshape(2, N, ROW_W)

    return _combine(parts, h, a_s, a_d, c, bias.reshape(1, C))


# speedup vs baseline: 11.4064x; 11.4064x over previous
import functools

import jax
import jax.numpy as jnp
from jax import lax
from jax.experimental import pallas as pl
from jax.experimental.pallas import tpu as pltpu
from jax.experimental.pallas import tpu_sc as plsc

N = 10000
E = 320000
C = 128
NEG = 0.2

TILES = 16
B = 128
CH = 8
NCH = 20
NB = NCH * CH
EPT = NB * B
PAD = TILES * EPT - E
ACC_ROWS = 10240
DEN_BASE = 10048
DEN_ROWS = 79
NPAD = 10016
BLK = 1000


def _prologue_body(x_ref, w_ref, asrc_ref, adst_ref,
                   h_ref, as_ref, ad_ref, c_ref, m_ref):
    i = pl.program_id(0)
    h = jnp.dot(x_ref[...], w_ref[...], preferred_element_type=jnp.float32)
    h_ref[...] = h
    a_s = jnp.sum(h * asrc_ref[...], axis=1, keepdims=True)
    a_d = jnp.sum(h * adst_ref[...], axis=1, keepdims=True)
    as_ref[...] = a_s
    ad_ref[...] = a_d
    bs = jnp.max(a_s)
    bd = jnp.max(a_d)

    @pl.when(i == 0)
    def _():
        m_ref[0] = bs
        m_ref[1] = bd

    @pl.when(i > 0)
    def _():
        m_ref[0] = jnp.maximum(m_ref[0], bs)
        m_ref[1] = jnp.maximum(m_ref[1], bd)

    @pl.when(i == N // BLK - 1)
    def _():
        s = m_ref[0] + m_ref[1]
        c_ref[...] = jnp.maximum(s, NEG * s).reshape(1, 1)


_prologue = pl.pallas_call(
    _prologue_body,
    grid=(N // BLK,),
    in_specs=[
        pl.BlockSpec((BLK, C), lambda i: (i, 0)),
        pl.BlockSpec((C, C), lambda i: (0, 0)),
        pl.BlockSpec((1, C), lambda i: (0, 0)),
        pl.BlockSpec((1, C), lambda i: (0, 0)),
    ],
    out_specs=[
        pl.BlockSpec((BLK, C), lambda i: (i, 0)),
        pl.BlockSpec((BLK, 1), lambda i: (i, 0)),
        pl.BlockSpec((BLK, 1), lambda i: (i, 0)),
        pl.BlockSpec((1, 1), lambda i: (0, 0)),
    ],
    out_shape=[
        jax.ShapeDtypeStruct((N, C), jnp.float32),
        jax.ShapeDtypeStruct((N, 1), jnp.float32),
        jax.ShapeDtypeStruct((N, 1), jnp.float32),
        jax.ShapeDtypeStruct((1, 1), jnp.float32),
    ],
    scratch_shapes=[pltpu.SMEM((2,), jnp.float32)],
)


_sc_mesh = plsc.VectorSubcoreMesh(
    core_axis_name="c", subcore_axis_name="s", num_cores=1, num_subcores=16)


@functools.partial(
    pl.kernel,
    out_type=jax.ShapeDtypeStruct((ACC_ROWS, C), jnp.float32),
    mesh=_sc_mesh,
    compiler_params=pltpu.CompilerParams(needs_layout_passes=False),
    scratch_types=[
        pltpu.VMEM((NPAD,), jnp.float32),
        pltpu.VMEM((NPAD,), jnp.float32),
        pltpu.VMEM((16,), jnp.float32),
        pltpu.VMEM((CH, B), jnp.int32),
        pltpu.VMEM((CH, B), jnp.int32),
        pltpu.VMEM((4, 32), jnp.int32),
        pltpu.VMEM((B, C), jnp.float32),
        pltpu.VMEM((32, C), jnp.float32),
        pltpu.VMEM((B,), jnp.float32),
        pltpu.VMEM_SHARED((ACC_ROWS, C), jnp.float32),
        pltpu.SemaphoreType.DMA,
    ],
)
def _edge_pass(h_hbm, as_hbm, ad_hbm, c_hbm, src_hbm, dst_hbm, out_hbm,
               as_v, ad_v, c_v, srcv, dstv, dixv, rows_v, den_v,
               wbuf, acc, sem):
    sub = lax.axis_index("s")
    tile = sub

    pltpu.sync_copy(as_hbm, as_v)
    pltpu.sync_copy(ad_hbm, ad_v)
    pltpu.sync_copy(c_hbm, c_v)

    zero16 = jnp.zeros((16,), jnp.float32)

    @pl.loop(0, 32)
    def _(i):
        @pl.loop(0, C // 16)
        def _(j):
            den_v[i, pl.ds(j * 16, 16)] = zero16

    @pl.loop(0, ACC_ROWS // (16 * 32))
    def _(k):
        pltpu.sync_copy(den_v, acc.at[pl.ds(sub * 640 + k * 32, 32)])

    plsc.subcore_barrier()

    iota16 = lax.iota(jnp.int32, 16)
    cvec = c_v[...]

    @pl.loop(0, NCH)
    def _(ch):
        pltpu.sync_copy(src_hbm.at[tile, pl.ds(ch * CH, CH)], srcv)
        pltpu.sync_copy(dst_hbm.at[tile, pl.ds(ch * CH, CH)], dstv)

        @pl.loop(0, CH)
        def _(b):
            gcp = pltpu.async_copy(h_hbm.at[srcv.at[b]], rows_v, sem)

            @pl.loop(0, B // 16)
            def _(g):
                sidx = srcv[b, pl.ds(g * 16, 16)]
                didx = dstv[b, pl.ds(g * 16, 16)]
                e = (plsc.load_gather(as_v, [sidx])
                     + plsc.load_gather(ad_v, [didx]))
                e = jnp.maximum(e, NEG * e)
                w = jnp.exp(e - cvec)
                wbuf[pl.ds(g * 16, 16)] = w
                dixv[g >> 1, pl.ds((g & 1) * 16, 16)] = DEN_BASE + (didx >> 7)

            gcp.wait()

            @pl.loop(0, B // 16)
            def _(g):
                wvec = wbuf[pl.ds(g * 16, 16)]
                for jj in range(16):
                    i = g * 16 + jj
                    ws = wvec[jj]
                    for j in range(C // 16):
                        rows_v[i, pl.ds(j * 16, 16)] = (
                            rows_v[i, pl.ds(j * 16, 16)] * ws)

            pltpu.sync_copy(rows_v, acc.at[dstv.at[b]], add=True)

            @pl.loop(0, 4)
            def _(s):
                for half in range(2):
                    g16 = s * 32 + half * 16
                    rowv = iota16 + half * 16
                    didx = dstv[b, pl.ds(g16, 16)]
                    w = wbuf[pl.ds(g16, 16)]
                    plsc.store_scatter(den_v, [rowv, didx & 127], w)
                pltpu.sync_copy(den_v, acc.at[dixv.at[s]], add=True)
                for half in range(2):
                    g16 = s * 32 + half * 16
                    rowv = iota16 + half * 16
                    didx = dstv[b, pl.ds(g16, 16)]
                    plsc.store_scatter(den_v, [rowv, didx & 127], zero16)

    plsc.subcore_barrier()

    pltpu.sync_copy(acc.at[pl.ds(sub * 640, 640)],
                    out_hbm.at[pl.ds(sub * 640, 640)])


def _combine_body(p_ref, d_ref, h_ref, as_ref, ad_ref, c_ref, b_ref, o_ref):
    es = as_ref[...] + ad_ref[...]
    es = jnp.maximum(es, NEG * es)
    ws = jnp.exp(es - c_ref[0, 0])
    num = p_ref[...] + ws * h_ref[...]
    den = d_ref[...] + ws
    o_ref[...] = num / (den + 1e-16) + b_ref[...]


_combine = pl.pallas_call(
    _combine_body,
    grid=(N // BLK,),
    in_specs=[
        pl.BlockSpec((BLK, C), lambda i: (i, 0)),
        pl.BlockSpec((BLK, 1), lambda i: (i, 0)),
        pl.BlockSpec((BLK, C), lambda i: (i, 0)),
        pl.BlockSpec((BLK, 1), lambda i: (i, 0)),
        pl.BlockSpec((BLK, 1), lambda i: (i, 0)),
        pl.BlockSpec((1, 1), lambda i: (0, 0)),
        pl.BlockSpec((1, C), lambda i: (0, 0)),
    ],
    out_specs=pl.BlockSpec((BLK, C), lambda i: (i, 0)),
    out_shape=jax.ShapeDtypeStruct((N, C), jnp.float32),
)


def kernel(x, edge_index, W, att_src, att_dst, bias):
    src = edge_index[0].astype(jnp.int32)
    dst = edge_index[1].astype(jnp.int32)
    src_p = jnp.concatenate(
        [src, jnp.zeros((PAD,), jnp.int32)]).reshape(TILES, NB, B)
    dst_p = jnp.concatenate(
        [dst, jnp.full((PAD,), N, jnp.int32)]).reshape(TILES, NB, B)

    h, a_s, a_d, c = _prologue(
        x, W, att_src.reshape(1, C), att_dst.reshape(1, C))
    cvec = jnp.broadcast_to(c[0], (16,))

    zpad = jnp.zeros((NPAD - N,), jnp.float32)
    as_p = jnp.concatenate([a_s.reshape(N), zpad])
    ad_p = jnp.concatenate([a_d.reshape(N), zpad])

    parts = _edge_pass(h, as_p, ad_p, cvec, src_p, dst_p)
    den = parts[DEN_BASE:DEN_BASE + DEN_ROWS, :].reshape(
        DEN_ROWS * C)[:N].reshape(N, 1)

    return _combine(parts, den, h, a_s, a_d, c, bias.reshape(1, C))


# speedup vs baseline: 11.8924x; 1.0426x over previous
import functools

import jax
import jax.numpy as jnp
from jax import lax
from jax.experimental import pallas as pl
from jax.experimental.pallas import tpu as pltpu
from jax.experimental.pallas import tpu_sc as plsc

N = 10000
E = 320000
C = 128
NEG = 0.2

TILES = 16
B = 128
CH = 8
NCH = 20
NB = NCH * CH
EPT = NB * B
PAD = TILES * EPT - E
ACC_ROWS = 10240
DEN_BASE = 10048
DEN_ROWS = 79
NPAD = 10016
BLK = 1000


def _prologue_body(x_ref, w_ref, asrc_ref, adst_ref,
                   h_ref, as_ref, ad_ref, c_ref, m_ref):
    i = pl.program_id(0)
    h = jnp.dot(x_ref[...], w_ref[...], preferred_element_type=jnp.float32)
    h_ref[...] = h
    a_s = jnp.sum(h * asrc_ref[...], axis=1, keepdims=True)
    a_d = jnp.sum(h * adst_ref[...], axis=1, keepdims=True)
    as_ref[...] = a_s
    ad_ref[...] = a_d
    bs = jnp.max(a_s)
    bd = jnp.max(a_d)

    @pl.when(i == 0)
    def _():
        m_ref[0] = bs
        m_ref[1] = bd

    @pl.when(i > 0)
    def _():
        m_ref[0] = jnp.maximum(m_ref[0], bs)
        m_ref[1] = jnp.maximum(m_ref[1], bd)

    @pl.when(i == N // BLK - 1)
    def _():
        s = m_ref[0] + m_ref[1]
        c_ref[...] = jnp.maximum(s, NEG * s).reshape(1, 1)


_prologue = pl.pallas_call(
    _prologue_body,
    grid=(N // BLK,),
    in_specs=[
        pl.BlockSpec((BLK, C), lambda i: (i, 0)),
        pl.BlockSpec((C, C), lambda i: (0, 0)),
        pl.BlockSpec((1, C), lambda i: (0, 0)),
        pl.BlockSpec((1, C), lambda i: (0, 0)),
    ],
    out_specs=[
        pl.BlockSpec((BLK, C), lambda i: (i, 0)),
        pl.BlockSpec((BLK, 1), lambda i: (i, 0)),
        pl.BlockSpec((BLK, 1), lambda i: (i, 0)),
        pl.BlockSpec((1, 1), lambda i: (0, 0)),
    ],
    out_shape=[
        jax.ShapeDtypeStruct((N, C), jnp.float32),
        jax.ShapeDtypeStruct((N, 1), jnp.float32),
        jax.ShapeDtypeStruct((N, 1), jnp.float32),
        jax.ShapeDtypeStruct((1, 1), jnp.float32),
    ],
    scratch_shapes=[pltpu.SMEM((2,), jnp.float32)],
)


_sc_mesh = plsc.VectorSubcoreMesh(
    core_axis_name="c", subcore_axis_name="s", num_cores=1, num_subcores=16)


@functools.partial(
    pl.kernel,
    out_type=jax.ShapeDtypeStruct((ACC_ROWS, C), jnp.float32),
    mesh=_sc_mesh,
    compiler_params=pltpu.CompilerParams(needs_layout_passes=False),
    scratch_types=[
        pltpu.VMEM((NPAD,), jnp.float32),
        pltpu.VMEM((NPAD,), jnp.float32),
        pltpu.VMEM((16,), jnp.float32),
        pltpu.VMEM((CH, B), jnp.int32),
        pltpu.VMEM((CH, B), jnp.int32),
        pltpu.VMEM((4, 32), jnp.int32),
        pltpu.VMEM((B, C), jnp.float32),
        pltpu.VMEM((32, C), jnp.float32),
        pltpu.VMEM((B,), jnp.float32),
        pltpu.VMEM_SHARED((ACC_ROWS, C), jnp.float32),
        pltpu.SemaphoreType.DMA,
        pltpu.SemaphoreType.DMA,
    ],
)
def _edge_pass(h_hbm, as_hbm, ad_hbm, c_hbm, src_hbm, dst_hbm, out_hbm,
               as_v, ad_v, c_v, srcv, dstv, dixv, rows_v, den_v,
               wbuf, acc, sem, sem_m):
    sub = lax.axis_index("s")
    tile = sub

    pltpu.sync_copy(as_hbm, as_v)
    pltpu.sync_copy(ad_hbm, ad_v)
    pltpu.sync_copy(c_hbm, c_v)

    zero16 = jnp.zeros((16,), jnp.float32)

    @pl.loop(0, 32)
    def _(i):
        @pl.loop(0, C // 16)
        def _(j):
            den_v[i, pl.ds(j * 16, 16)] = zero16

    @pl.loop(0, ACC_ROWS // (16 * 32))
    def _(k):
        pltpu.sync_copy(den_v, acc.at[pl.ds(sub * 640 + k * 32, 32)])

    plsc.subcore_barrier()

    iota16 = lax.iota(jnp.int32, 16)
    cvec = c_v[...]

    @pl.loop(0, NCH)
    def _(ch):
        pltpu.sync_copy(src_hbm.at[tile, pl.ds(ch * CH, CH)], srcv)
        pltpu.sync_copy(dst_hbm.at[tile, pl.ds(ch * CH, CH)], dstv)

        @pl.loop(0, CH)
        def _(b):
            gcp = pltpu.async_copy(h_hbm.at[srcv.at[b]], rows_v, sem)

            @pl.loop(0, B // 16)
            def _(g):
                sidx = srcv[b, pl.ds(g * 16, 16)]
                didx = dstv[b, pl.ds(g * 16, 16)]
                e = (plsc.load_gather(as_v, [sidx])
                     + plsc.load_gather(ad_v, [didx]))
                e = jnp.maximum(e, NEG * e)
                w = jnp.exp(e - cvec)
                wbuf[pl.ds(g * 16, 16)] = w
                dixv[g >> 1, pl.ds((g & 1) * 16, 16)] = DEN_BASE + (didx >> 7)

            gcp.wait()

            @pl.loop(0, B // 16)
            def _(g):
                wvec = wbuf[pl.ds(g * 16, 16)]
                for jj in range(16):
                    i = g * 16 + jj
                    ws = wvec[jj]
                    for j in range(C // 16):
                        rows_v[i, pl.ds(j * 16, 16)] = (
                            rows_v[i, pl.ds(j * 16, 16)] * ws)

            mcp = pltpu.async_copy(rows_v, acc.at[dstv.at[b]], sem_m,
                                   add=True)

            @pl.loop(0, 4)
            def _(s):
                for half in range(2):
                    g16 = s * 32 + half * 16
                    rowv = iota16 + half * 16
                    didx = dstv[b, pl.ds(g16, 16)]
                    w = wbuf[pl.ds(g16, 16)]
                    plsc.store_scatter(den_v, [rowv, didx & 127], w)
                pltpu.sync_copy(den_v, acc.at[dixv.at[s]], add=True)
                for half in range(2):
                    g16 = s * 32 + half * 16
                    rowv = iota16 + half * 16
                    didx = dstv[b, pl.ds(g16, 16)]
                    plsc.store_scatter(den_v, [rowv, didx & 127], zero16)

            mcp.wait()

    plsc.subcore_barrier()

    pltpu.sync_copy(acc.at[pl.ds(sub * 640, 640)],
                    out_hbm.at[pl.ds(sub * 640, 640)])


def _combine_body(p_ref, d_ref, h_ref, as_ref, ad_ref, c_ref, b_ref, o_ref):
    es = as_ref[...] + ad_ref[...]
    es = jnp.maximum(es, NEG * es)
    ws = jnp.exp(es - c_ref[0, 0])
    num = p_ref[...] + ws * h_ref[...]
    den = d_ref[...] + ws
    o_ref[...] = num / (den + 1e-16) + b_ref[...]


_combine = pl.pallas_call(
    _combine_body,
    grid=(N // BLK,),
    in_specs=[
        pl.BlockSpec((BLK, C), lambda i: (i, 0)),
        pl.BlockSpec((BLK, 1), lambda i: (i, 0)),
        pl.BlockSpec((BLK, C), lambda i: (i, 0)),
        pl.BlockSpec((BLK, 1), lambda i: (i, 0)),
        pl.BlockSpec((BLK, 1), lambda i: (i, 0)),
        pl.BlockSpec((1, 1), lambda i: (0, 0)),
        pl.BlockSpec((1, C), lambda i: (0, 0)),
    ],
    out_specs=pl.BlockSpec((BLK, C), lambda i: (i, 0)),
    out_shape=jax.ShapeDtypeStruct((N, C), jnp.float32),
)


def kernel(x, edge_index, W, att_src, att_dst, bias):
    src = edge_index[0].astype(jnp.int32)
    dst = edge_index[1].astype(jnp.int32)
    src_p = jnp.concatenate(
        [src, jnp.zeros((PAD,), jnp.int32)]).reshape(TILES, NB, B)
    dst_p = jnp.concatenate(
        [dst, jnp.full((PAD,), N, jnp.int32)]).reshape(TILES, NB, B)

    h, a_s, a_d, c = _prologue(
        x, W, att_src.reshape(1, C), att_dst.reshape(1, C))
    cvec = jnp.broadcast_to(c[0], (16,))

    zpad = jnp.zeros((NPAD - N,), jnp.float32)
    as_p = jnp.concatenate([a_s.reshape(N), zpad])
    ad_p = jnp.concatenate([a_d.reshape(N), zpad])

    parts = _edge_pass(h, as_p, ad_p, cvec, src_p, dst_p)
    den = parts[DEN_BASE:DEN_BASE + DEN_ROWS, :].reshape(
        DEN_ROWS * C)[:N].reshape(N, 1)

    return _combine(parts, den, h, a_s, a_d, c, bias.reshape(1, C))


# speedup vs baseline: 15.3024x; 1.2867x over previous
import functools

import jax
import jax.numpy as jnp
from jax import lax
from jax.experimental import pallas as pl
from jax.experimental.pallas import tpu as pltpu
from jax.experimental.pallas import tpu_sc as plsc

N = 10000
E = 320000
C = 128
NEG = 0.2

TILES = 16
B = 32
CH = 32
NCH = 20
NB = NCH * CH
EPT = NB * B
PAD = TILES * EPT - E
ACC_ROWS = 10240
DEN_BASE = 10048
DEN_ROWS = 79
DEN_PAD = 80
NPAD = 10016
BLK = 1000


def _prologue_body(x_ref, w_ref, asrc_ref, adst_ref,
                   h_ref, as_ref, ad_ref, c_ref, m_ref):
    i = pl.program_id(0)
    h = jnp.dot(x_ref[...], w_ref[...], preferred_element_type=jnp.float32)
    h_ref[...] = h
    a_s = jnp.sum(h * asrc_ref[...], axis=1, keepdims=True)
    a_d = jnp.sum(h * adst_ref[...], axis=1, keepdims=True)
    as_ref[...] = a_s
    ad_ref[...] = a_d
    bs = jnp.max(a_s)
    bd = jnp.max(a_d)

    @pl.when(i == 0)
    def _():
        m_ref[0] = bs
        m_ref[1] = bd

    @pl.when(i > 0)
    def _():
        m_ref[0] = jnp.maximum(m_ref[0], bs)
        m_ref[1] = jnp.maximum(m_ref[1], bd)

    @pl.when(i == N // BLK - 1)
    def _():
        s = m_ref[0] + m_ref[1]
        c_ref[...] = jnp.maximum(s, NEG * s).reshape(1, 1)


_prologue = pl.pallas_call(
    _prologue_body,
    grid=(N // BLK,),
    in_specs=[
        pl.BlockSpec((BLK, C), lambda i: (i, 0)),
        pl.BlockSpec((C, C), lambda i: (0, 0)),
        pl.BlockSpec((1, C), lambda i: (0, 0)),
        pl.BlockSpec((1, C), lambda i: (0, 0)),
    ],
    out_specs=[
        pl.BlockSpec((BLK, C), lambda i: (i, 0)),
        pl.BlockSpec((BLK, 1), lambda i: (i, 0)),
        pl.BlockSpec((BLK, 1), lambda i: (i, 0)),
        pl.BlockSpec((1, 1), lambda i: (0, 0)),
    ],
    out_shape=[
        jax.ShapeDtypeStruct((N, C), jnp.float32),
        jax.ShapeDtypeStruct((N, 1), jnp.float32),
        jax.ShapeDtypeStruct((N, 1), jnp.float32),
        jax.ShapeDtypeStruct((1, 1), jnp.float32),
    ],
    scratch_shapes=[pltpu.SMEM((2,), jnp.float32)],
)


_sc_mesh = plsc.VectorSubcoreMesh(
    core_axis_name="c", subcore_axis_name="s", num_cores=1, num_subcores=16)


@functools.partial(
    pl.kernel,
    out_type=jax.ShapeDtypeStruct((ACC_ROWS, C), jnp.float32),
    mesh=_sc_mesh,
    compiler_params=pltpu.CompilerParams(needs_layout_passes=False),
    scratch_types=[
        pltpu.VMEM((NPAD,), jnp.float32),
        pltpu.VMEM((NPAD,), jnp.float32),
        pltpu.VMEM((16,), jnp.float32),
        pltpu.VMEM((CH, B), jnp.int32),
        pltpu.VMEM((CH, B), jnp.int32),
        pltpu.VMEM((2 * B, C), jnp.float32),
        pltpu.VMEM((DEN_PAD, C), jnp.float32),
        pltpu.VMEM((DEN_PAD,), jnp.int32),
        pltpu.VMEM((2 * B,), jnp.float32),
        pltpu.VMEM_SHARED((ACC_ROWS, C), jnp.float32),
        pltpu.SemaphoreType.DMA,
        pltpu.SemaphoreType.DMA,
        pltpu.SemaphoreType.DMA,
        pltpu.SemaphoreType.DMA,
    ],
)
def _edge_pass(h_hbm, as_hbm, ad_hbm, c_hbm, src_hbm, dst_hbm, out_hbm,
               as_v, ad_v, c_v, srcv, dstv, rows_v, den_v, dix_v,
               wbuf, acc, sem_g0, sem_g1, sem_m0, sem_m1):
    sub = lax.axis_index("s")
    tile = sub

    pltpu.sync_copy(as_hbm, as_v)
    pltpu.sync_copy(ad_hbm, ad_v)
    pltpu.sync_copy(c_hbm, c_v)

    zero16 = jnp.zeros((16,), jnp.float32)
    iota16 = lax.iota(jnp.int32, 16)

    @pl.loop(0, DEN_PAD // 16)
    def _(k):
        dix_v[pl.ds(k * 16, 16)] = DEN_BASE + k * 16 + iota16

    @pl.loop(0, DEN_PAD)
    def _(i):
        @pl.loop(0, C // 16)
        def _(j):
            den_v[i, pl.ds(j * 16, 16)] = zero16

    @pl.loop(0, ACC_ROWS // (16 * 32))
    def _(k):
        pltpu.sync_copy(den_v.at[pl.ds(0, 32)],
                        acc.at[pl.ds(sub * 640 + k * 32, 32)])

    plsc.subcore_barrier()

    cvec = c_v[...]

    def _weights(b):
        for g in range(B // 16):
            sidx = srcv[b, pl.ds(g * 16, 16)]
            didx = dstv[b, pl.ds(g * 16, 16)]
            e = (plsc.load_gather(as_v, [sidx])
                 + plsc.load_gather(ad_v, [didx]))
            e = jnp.maximum(e, NEG * e)
            w = jnp.exp(e - cvec)
            wbuf[pl.ds((b & 1) * B + g * 16, 16)] = w
            plsc.addupdate_scatter(den_v, [didx >> 7, didx & 127], w)

    def _scale(h):
        for g in range(B // 16):
            wvec = wbuf[pl.ds(h * B + g * 16, 16)]
            for jj in range(16):
                i = h * B + g * 16 + jj
                ws = wvec[jj]
                for j in range(C // 16):
                    rows_v[i, pl.ds(j * 16, 16)] = (
                        rows_v[i, pl.ds(j * 16, 16)] * ws)

    @pl.loop(0, NCH)
    def _(ch):
        pltpu.sync_copy(src_hbm.at[tile, ch], srcv)
        pltpu.sync_copy(dst_hbm.at[tile, ch], dstv)

        @pl.loop(0, CH // 2)
        def _(p):
            b0 = 2 * p
            b1 = 2 * p + 1
            g0 = pltpu.async_copy(h_hbm.at[srcv.at[b0]],
                                  rows_v.at[pl.ds(0, B)], sem_g0)
            g1 = pltpu.async_copy(h_hbm.at[srcv.at[b1]],
                                  rows_v.at[pl.ds(B, B)], sem_g1)
            _weights(b0)
            _weights(b1)
            g0.wait()
            _scale(0)
            m0 = pltpu.async_copy(rows_v.at[pl.ds(0, B)],
                                  acc.at[dstv.at[b0]], sem_m0, add=True)
            g1.wait()
            _scale(1)
            m1 = pltpu.async_copy(rows_v.at[pl.ds(B, B)],
                                  acc.at[dstv.at[b1]], sem_m1, add=True)
            m0.wait()
            m1.wait()

    pltpu.sync_copy(den_v, acc.at[dix_v.at[...]], add=True)

    plsc.subcore_barrier()

    pltpu.sync_copy(acc.at[pl.ds(sub * 640, 640)],
                    out_hbm.at[pl.ds(sub * 640, 640)])


def _combine_body(p_ref, d_ref, h_ref, as_ref, ad_ref, c_ref, b_ref, o_ref):
    es = as_ref[...] + ad_ref[...]
    es = jnp.maximum(es, NEG * es)
    ws = jnp.exp(es - c_ref[0, 0])
    num = p_ref[...] + ws * h_ref[...]
    den = d_ref[...] + ws
    o_ref[...] = num / (den + 1e-16) + b_ref[...]


_combine = pl.pallas_call(
    _combine_body,
    grid=(N // BLK,),
    in_specs=[
        pl.BlockSpec((BLK, C), lambda i: (i, 0)),
        pl.BlockSpec((BLK, 1), lambda i: (i, 0)),
        pl.BlockSpec((BLK, C), lambda i: (i, 0)),
        pl.BlockSpec((BLK, 1), lambda i: (i, 0)),
        pl.BlockSpec((BLK, 1), lambda i: (i, 0)),
        pl.BlockSpec((1, 1), lambda i: (0, 0)),
        pl.BlockSpec((1, C), lambda i: (0, 0)),
    ],
    out_specs=pl.BlockSpec((BLK, C), lambda i: (i, 0)),
    out_shape=jax.ShapeDtypeStruct((N, C), jnp.float32),
)


def kernel(x, edge_index, W, att_src, att_dst, bias):
    src = edge_index[0].astype(jnp.int32)
    dst = edge_index[1].astype(jnp.int32)
    src_p = jnp.concatenate(
        [src, jnp.zeros((PAD,), jnp.int32)]).reshape(TILES, NCH, CH, B)
    dst_p = jnp.concatenate(
        [dst, jnp.full((PAD,), N, jnp.int32)]).reshape(TILES, NCH, CH, B)

    h, a_s, a_d, c = _prologue(
        x, W, att_src.reshape(1, C), att_dst.reshape(1, C))
    cvec = jnp.broadcast_to(c[0], (16,))

    zpad = jnp.zeros((NPAD - N,), jnp.float32)
    as_p = jnp.concatenate([a_s.reshape(N), zpad])
    ad_p = jnp.concatenate([a_d.reshape(N), zpad])

    parts = _edge_pass(h, as_p, ad_p, cvec, src_p, dst_p)
    den = parts[DEN_BASE:DEN_BASE + DEN_ROWS, :].reshape(
        DEN_ROWS * C)[:N].reshape(N, 1)

    return _combine(parts, den, h, a_s, a_d, c, bias.reshape(1, C))


# speedup vs baseline: 19.6257x; 1.2825x over previous
import functools

import jax
import jax.numpy as jnp
from jax import lax
from jax.experimental import pallas as pl
from jax.experimental.pallas import tpu as pltpu
from jax.experimental.pallas import tpu_sc as plsc

N = 10000
E = 320000
C = 128
NEG = 0.2

TILES = 32
B = 32
CH = 32
NCH = 10
NB = NCH * CH
EPT = NB * B
PAD = TILES * EPT - E
ACC_ROWS = 10240
DEN_BASE = 10048
DEN_ROWS = 79
DEN_PAD = 80
NPAD = 10016
BLK = 1000


def _prologue_body(x_ref, w_ref, asrc_ref, adst_ref,
                   h_ref, as_ref, ad_ref, c_ref, m_ref):
    i = pl.program_id(0)
    h = jnp.dot(x_ref[...], w_ref[...], preferred_element_type=jnp.float32)
    h_ref[...] = h
    a_s = jnp.sum(h * asrc_ref[...], axis=1, keepdims=True)
    a_d = jnp.sum(h * adst_ref[...], axis=1, keepdims=True)
    as_ref[...] = a_s
    ad_ref[...] = a_d
    bs = jnp.max(a_s)
    bd = jnp.max(a_d)

    @pl.when(i == 0)
    def _():
        m_ref[0] = bs
        m_ref[1] = bd

    @pl.when(i > 0)
    def _():
        m_ref[0] = jnp.maximum(m_ref[0], bs)
        m_ref[1] = jnp.maximum(m_ref[1], bd)

    @pl.when(i == N // BLK - 1)
    def _():
        s = m_ref[0] + m_ref[1]
        c_ref[...] = jnp.maximum(s, NEG * s).reshape(1, 1)


_prologue = pl.pallas_call(
    _prologue_body,
    grid=(N // BLK,),
    in_specs=[
        pl.BlockSpec((BLK, C), lambda i: (i, 0)),
        pl.BlockSpec((C, C), lambda i: (0, 0)),
        pl.BlockSpec((1, C), lambda i: (0, 0)),
        pl.BlockSpec((1, C), lambda i: (0, 0)),
    ],
    out_specs=[
        pl.BlockSpec((BLK, C), lambda i: (i, 0)),
        pl.BlockSpec((BLK, 1), lambda i: (i, 0)),
        pl.BlockSpec((BLK, 1), lambda i: (i, 0)),
        pl.BlockSpec((1, 1), lambda i: (0, 0)),
    ],
    out_shape=[
        jax.ShapeDtypeStruct((N, C), jnp.float32),
        jax.ShapeDtypeStruct((N, 1), jnp.float32),
        jax.ShapeDtypeStruct((N, 1), jnp.float32),
        jax.ShapeDtypeStruct((1, 1), jnp.float32),
    ],
    scratch_shapes=[pltpu.SMEM((2,), jnp.float32)],
)


_sc_mesh = plsc.VectorSubcoreMesh(
    core_axis_name="c", subcore_axis_name="s", num_cores=2, num_subcores=16)


@functools.partial(
    pl.kernel,
    out_type=jax.ShapeDtypeStruct((2, ACC_ROWS, C), jnp.float32),
    mesh=_sc_mesh,
    compiler_params=pltpu.CompilerParams(needs_layout_passes=False),
    scratch_types=[
        pltpu.VMEM((NPAD,), jnp.float32),
        pltpu.VMEM((NPAD,), jnp.float32),
        pltpu.VMEM((16,), jnp.float32),
        pltpu.VMEM((CH, B), jnp.int32),
        pltpu.VMEM((CH, B), jnp.int32),
        pltpu.VMEM((2 * B, C), jnp.float32),
        pltpu.VMEM((DEN_PAD, C), jnp.float32),
        pltpu.VMEM((DEN_PAD,), jnp.int32),
        pltpu.VMEM((2 * B,), jnp.float32),
        pltpu.VMEM_SHARED((ACC_ROWS, C), jnp.float32),
        pltpu.SemaphoreType.DMA,
        pltpu.SemaphoreType.DMA,
        pltpu.SemaphoreType.DMA,
        pltpu.SemaphoreType.DMA,
    ],
)
def _edge_pass(h_hbm, as_hbm, ad_hbm, c_hbm, src_hbm, dst_hbm, out_hbm,
               as_v, ad_v, c_v, srcv, dstv, rows_v, den_v, dix_v,
               wbuf, acc, sem_g0, sem_g1, sem_m0, sem_m1):
    sub = lax.axis_index("s")
    core = lax.axis_index("c")
    tile = core * 16 + sub

    pltpu.sync_copy(as_hbm, as_v)
    pltpu.sync_copy(ad_hbm, ad_v)
    pltpu.sync_copy(c_hbm, c_v)

    zero16 = jnp.zeros((16,), jnp.float32)
    iota16 = lax.iota(jnp.int32, 16)

    @pl.loop(0, DEN_PAD // 16)
    def _(k):
        dix_v[pl.ds(k * 16, 16)] = DEN_BASE + k * 16 + iota16

    @pl.loop(0, DEN_PAD)
    def _(i):
        @pl.loop(0, C // 16)
        def _(j):
            den_v[i, pl.ds(j * 16, 16)] = zero16

    @pl.loop(0, ACC_ROWS // (16 * 32))
    def _(k):
        pltpu.sync_copy(den_v.at[pl.ds(0, 32)],
                        acc.at[pl.ds(sub * 640 + k * 32, 32)])

    plsc.subcore_barrier()

    cvec = c_v[...]

    def _weights(b):
        for g in range(B // 16):
            sidx = srcv[b, pl.ds(g * 16, 16)]
            didx = dstv[b, pl.ds(g * 16, 16)]
            e = (plsc.load_gather(as_v, [sidx])
                 + plsc.load_gather(ad_v, [didx]))
            e = jnp.maximum(e, NEG * e)
            w = jnp.exp(e - cvec)
            wbuf[pl.ds((b & 1) * B + g * 16, 16)] = w
            plsc.addupdate_scatter(den_v, [didx >> 7, didx & 127], w)

    def _scale(h):
        for g in range(B // 16):
            wvec = wbuf[pl.ds(h * B + g * 16, 16)]
            for jj in range(16):
                i = h * B + g * 16 + jj
                ws = wvec[jj]
                for j in range(C // 16):
                    rows_v[i, pl.ds(j * 16, 16)] = (
                        rows_v[i, pl.ds(j * 16, 16)] * ws)

    @pl.loop(0, NCH)
    def _(ch):
        pltpu.sync_copy(src_hbm.at[tile, ch], srcv)
        pltpu.sync_copy(dst_hbm.at[tile, ch], dstv)

        @pl.loop(0, CH // 2)
        def _(p):
            b0 = 2 * p
            b1 = 2 * p + 1
            g0 = pltpu.async_copy(h_hbm.at[srcv.at[b0]],
                                  rows_v.at[pl.ds(0, B)], sem_g0)
            g1 = pltpu.async_copy(h_hbm.at[srcv.at[b1]],
                                  rows_v.at[pl.ds(B, B)], sem_g1)
            _weights(b0)
            _weights(b1)
            g0.wait()
            _scale(0)
            m0 = pltpu.async_copy(rows_v.at[pl.ds(0, B)],
                                  acc.at[dstv.at[b0]], sem_m0, add=True)
            g1.wait()
            _scale(1)
            m1 = pltpu.async_copy(rows_v.at[pl.ds(B, B)],
                                  acc.at[dstv.at[b1]], sem_m1, add=True)
            m0.wait()
            m1.wait()

    pltpu.sync_copy(den_v, acc.at[dix_v.at[...]], add=True)

    plsc.subcore_barrier()

    pltpu.sync_copy(acc.at[pl.ds(sub * 640, 640)],
                    out_hbm.at[core, pl.ds(sub * 640, 640)])


def _combine_body(p0_ref, p1_ref, d0_ref, d1_ref, h_ref, as_ref, ad_ref,
                  c_ref, b_ref, o_ref):
    es = as_ref[...] + ad_ref[...]
    es = jnp.maximum(es, NEG * es)
    ws = jnp.exp(es - c_ref[0, 0])
    num = p0_ref[...] + p1_ref[...] + ws * h_ref[...]
    den = d0_ref[...] + d1_ref[...] + ws
    o_ref[...] = num / (den + 1e-16) + b_ref[...]


_combine = pl.pallas_call(
    _combine_body,
    grid=(N // BLK,),
    in_specs=[
        pl.BlockSpec((BLK, C), lambda i: (i, 0)),
        pl.BlockSpec((BLK, C), lambda i: (i, 0)),
        pl.BlockSpec((BLK, 1), lambda i: (i, 0)),
        pl.BlockSpec((BLK, 1), lambda i: (i, 0)),
        pl.BlockSpec((BLK, C), lambda i: (i, 0)),
        pl.BlockSpec((BLK, 1), lambda i: (i, 0)),
        pl.BlockSpec((BLK, 1), lambda i: (i, 0)),
        pl.BlockSpec((1, 1), lambda i: (0, 0)),
        pl.BlockSpec((1, C), lambda i: (0, 0)),
    ],
    out_specs=pl.BlockSpec((BLK, C), lambda i: (i, 0)),
    out_shape=jax.ShapeDtypeStruct((N, C), jnp.float32),
)


def kernel(x, edge_index, W, att_src, att_dst, bias):
    src = edge_index[0].astype(jnp.int32)
    dst = edge_index[1].astype(jnp.int32)
    src_p = jnp.concatenate(
        [src, jnp.zeros((PAD,), jnp.int32)]).reshape(TILES, NCH, CH, B)
    dst_p = jnp.concatenate(
        [dst, jnp.full((PAD,), N, jnp.int32)]).reshape(TILES, NCH, CH, B)

    h, a_s, a_d, c = _prologue(
        x, W, att_src.reshape(1, C), att_dst.reshape(1, C))
    cvec = jnp.broadcast_to(c[0], (16,))

    zpad = jnp.zeros((NPAD - N,), jnp.float32)
    as_p = jnp.concatenate([a_s.reshape(N), zpad])
    ad_p = jnp.concatenate([a_d.reshape(N), zpad])

    parts = _edge_pass(h, as_p, ad_p, cvec, src_p, dst_p)
    den0 = parts[0, DEN_BASE:DEN_BASE + DEN_ROWS, :].reshape(
        DEN_ROWS * C)[:N].reshape(N, 1)
    den1 = parts[1, DEN_BASE:DEN_BASE + DEN_ROWS, :].reshape(
        DEN_ROWS * C)[:N].reshape(N, 1)

    return _combine(parts[0], parts[1], den0, den1, h, a_s, a_d, c,
                    bias.reshape(1, C))


# speedup vs baseline: 19.6318x; 1.0003x over previous
import functools

import jax
import jax.numpy as jnp
from jax import lax
from jax.experimental import pallas as pl
from jax.experimental.pallas import tpu as pltpu
from jax.experimental.pallas import tpu_sc as plsc

N = 10000
E = 320000
C = 128
NEG = 0.2

TILES = 32
B = 32
CH = 32
NCH = 10
NB = NCH * CH
EPT = NB * B
PAD = TILES * EPT - E
ACC_ROWS = 10240
DEN_BASE = 10048
DEN_ROWS = 79
DEN_PAD = 80
NPAD = 10016
BLK = 1000


def _prologue_body(x_ref, w_ref, asrc_ref, adst_ref,
                   h_ref, as_ref, ad_ref, c_ref, m_ref):
    i = pl.program_id(0)
    h = jnp.dot(x_ref[...], w_ref[...], preferred_element_type=jnp.float32)
    h_ref[...] = h
    a_s = jnp.sum(h * asrc_ref[...], axis=1, keepdims=True)
    a_d = jnp.sum(h * adst_ref[...], axis=1, keepdims=True)
    as_ref[...] = a_s
    ad_ref[...] = a_d
    bs = jnp.max(a_s)
    bd = jnp.max(a_d)

    @pl.when(i == 0)
    def _():
        m_ref[0] = bs
        m_ref[1] = bd

    @pl.when(i > 0)
    def _():
        m_ref[0] = jnp.maximum(m_ref[0], bs)
        m_ref[1] = jnp.maximum(m_ref[1], bd)

    @pl.when(i == N // BLK - 1)
    def _():
        s = m_ref[0] + m_ref[1]
        c_ref[...] = jnp.maximum(s, NEG * s).reshape(1, 1)


_prologue = pl.pallas_call(
    _prologue_body,
    grid=(N // BLK,),
    in_specs=[
        pl.BlockSpec((BLK, C), lambda i: (i, 0)),
        pl.BlockSpec((C, C), lambda i: (0, 0)),
        pl.BlockSpec((1, C), lambda i: (0, 0)),
        pl.BlockSpec((1, C), lambda i: (0, 0)),
    ],
    out_specs=[
        pl.BlockSpec((BLK, C), lambda i: (i, 0)),
        pl.BlockSpec((BLK, 1), lambda i: (i, 0)),
        pl.BlockSpec((BLK, 1), lambda i: (i, 0)),
        pl.BlockSpec((1, 1), lambda i: (0, 0)),
    ],
    out_shape=[
        jax.ShapeDtypeStruct((N, C), jnp.float32),
        jax.ShapeDtypeStruct((N, 1), jnp.float32),
        jax.ShapeDtypeStruct((N, 1), jnp.float32),
        jax.ShapeDtypeStruct((1, 1), jnp.float32),
    ],
    scratch_shapes=[pltpu.SMEM((2,), jnp.float32)],
)


_sc_mesh = plsc.VectorSubcoreMesh(
    core_axis_name="c", subcore_axis_name="s", num_cores=2, num_subcores=16)


@functools.partial(
    pl.kernel,
    out_type=jax.ShapeDtypeStruct((2, ACC_ROWS, C), jnp.float32),
    mesh=_sc_mesh,
    compiler_params=pltpu.CompilerParams(needs_layout_passes=False),
    scratch_types=[
        pltpu.VMEM((NPAD,), jnp.float32),
        pltpu.VMEM((NPAD,), jnp.float32),
        pltpu.VMEM((16,), jnp.float32),
        pltpu.VMEM((CH, B), jnp.int32),
        pltpu.VMEM((CH, B), jnp.int32),
        pltpu.VMEM((2 * B, C), jnp.float32),
        pltpu.VMEM((DEN_PAD, C), jnp.float32),
        pltpu.VMEM((DEN_PAD,), jnp.int32),
        pltpu.VMEM((2 * B,), jnp.float32),
        pltpu.VMEM_SHARED((ACC_ROWS, C), jnp.float32),
        pltpu.SemaphoreType.DMA,
        pltpu.SemaphoreType.DMA,
        pltpu.SemaphoreType.DMA,
        pltpu.SemaphoreType.DMA,
    ],
)
def _edge_pass(h_hbm, as_hbm, ad_hbm, c_hbm, src_hbm, dst_hbm, out_hbm,
               as_v, ad_v, c_v, srcv, dstv, rows_v, den_v, dix_v,
               wbuf, acc, sem_g0, sem_g1, sem_m0, sem_m1):
    sub = lax.axis_index("s")
    core = lax.axis_index("c")
    tile = core * 16 + sub

    pltpu.sync_copy(as_hbm, as_v)
    pltpu.sync_copy(ad_hbm, ad_v)
    pltpu.sync_copy(c_hbm, c_v)

    zero16 = jnp.zeros((16,), jnp.float32)
    iota16 = lax.iota(jnp.int32, 16)

    @pl.loop(0, DEN_PAD // 16)
    def _(k):
        dix_v[pl.ds(k * 16, 16)] = DEN_BASE + k * 16 + iota16

    @pl.loop(0, DEN_PAD)
    def _(i):
        @pl.loop(0, C // 16)
        def _(j):
            den_v[i, pl.ds(j * 16, 16)] = zero16

    @pl.loop(0, ACC_ROWS // (16 * 32))
    def _(k):
        pltpu.sync_copy(den_v.at[pl.ds(0, 32)],
                        acc.at[pl.ds(sub * 640 + k * 32, 32)])

    plsc.subcore_barrier()

    cvec = c_v[...]

    def _weights(b):
        for g in range(B // 16):
            sidx = srcv[b, pl.ds(g * 16, 16)]
            didx = dstv[b, pl.ds(g * 16, 16)]
            e = (plsc.load_gather(as_v, [sidx])
                 + plsc.load_gather(ad_v, [didx]))
            e = jnp.maximum(e, NEG * e)
            w = jnp.exp(e - cvec)
            wbuf[pl.ds((b & 1) * B + g * 16, 16)] = w

    def _den(b):
        for g in range(B // 16):
            didx = dstv[b, pl.ds(g * 16, 16)]
            w = wbuf[pl.ds((b & 1) * B + g * 16, 16)]
            plsc.addupdate_scatter(den_v, [didx >> 7, didx & 127], w)

    def _scale(h):
        for g in range(B // 16):
            wvec = wbuf[pl.ds(h * B + g * 16, 16)]
            for jj in range(16):
                i = h * B + g * 16 + jj
                ws = wvec[jj]
                for j in range(C // 16):
                    rows_v[i, pl.ds(j * 16, 16)] = (
                        rows_v[i, pl.ds(j * 16, 16)] * ws)

    @pl.loop(0, NCH)
    def _(ch):
        pltpu.sync_copy(src_hbm.at[tile, ch], srcv)
        pltpu.sync_copy(dst_hbm.at[tile, ch], dstv)

        @pl.loop(0, CH // 2)
        def _(p):
            b0 = 2 * p
            b1 = 2 * p + 1
            g0 = pltpu.async_copy(h_hbm.at[srcv.at[b0]],
                                  rows_v.at[pl.ds(0, B)], sem_g0)
            g1 = pltpu.async_copy(h_hbm.at[srcv.at[b1]],
                                  rows_v.at[pl.ds(B, B)], sem_g1)
            _weights(b0)
            _weights(b1)
            g0.wait()
            _scale(0)
            m0 = pltpu.async_copy(rows_v.at[pl.ds(0, B)],
                                  acc.at[dstv.at[b0]], sem_m0, add=True)
            g1.wait()
            _scale(1)
            m1 = pltpu.async_copy(rows_v.at[pl.ds(B, B)],
                                  acc.at[dstv.at[b1]], sem_m1, add=True)
            _den(b0)
            _den(b1)
            m0.wait()
            m1.wait()

    pltpu.sync_copy(den_v, acc.at[dix_v.at[...]], add=True)

    plsc.subcore_barrier()

    pltpu.sync_copy(acc.at[pl.ds(sub * 640, 640)],
                    out_hbm.at[core, pl.ds(sub * 640, 640)])


def _combine_body(p0_ref, p1_ref, d0_ref, d1_ref, h_ref, as_ref, ad_ref,
                  c_ref, b_ref, o_ref):
    es = as_ref[...] + ad_ref[...]
    es = jnp.maximum(es, NEG * es)
    ws = jnp.exp(es - c_ref[0, 0])
    num = p0_ref[...] + p1_ref[...] + ws * h_ref[...]
    den = d0_ref[...] + d1_ref[...] + ws
    o_ref[...] = num / (den + 1e-16) + b_ref[...]


_combine = pl.pallas_call(
    _combine_body,
    grid=(N // BLK,),
    in_specs=[
        pl.BlockSpec((BLK, C), lambda i: (i, 0)),
        pl.BlockSpec((BLK, C), lambda i: (i, 0)),
        pl.BlockSpec((BLK, 1), lambda i: (i, 0)),
        pl.BlockSpec((BLK, 1), lambda i: (i, 0)),
        pl.BlockSpec((BLK, C), lambda i: (i, 0)),
        pl.BlockSpec((BLK, 1), lambda i: (i, 0)),
        pl.BlockSpec((BLK, 1), lambda i: (i, 0)),
        pl.BlockSpec((1, 1), lambda i: (0, 0)),
        pl.BlockSpec((1, C), lambda i: (0, 0)),
    ],
    out_specs=pl.BlockSpec((BLK, C), lambda i: (i, 0)),
    out_shape=jax.ShapeDtypeStruct((N, C), jnp.float32),
)


def kernel(x, edge_index, W, att_src, att_dst, bias):
    src = edge_index[0].astype(jnp.int32)
    dst = edge_index[1].astype(jnp.int32)
    src_p = jnp.concatenate(
        [src, jnp.zeros((PAD,), jnp.int32)]).reshape(TILES, NCH, CH, B)
    dst_p = jnp.concatenate(
        [dst, jnp.full((PAD,), N, jnp.int32)]).reshape(TILES, NCH, CH, B)

    h, a_s, a_d, c = _prologue(
        x, W, att_src.reshape(1, C), att_dst.reshape(1, C))
    cvec = jnp.broadcast_to(c[0], (16,))

    zpad = jnp.zeros((NPAD - N,), jnp.float32)
    as_p = jnp.concatenate([a_s.reshape(N), zpad])
    ad_p = jnp.concatenate([a_d.reshape(N), zpad])

    parts = _edge_pass(h, as_p, ad_p, cvec, src_p, dst_p)
    den0 = parts[0, DEN_BASE:DEN_BASE + DEN_ROWS, :].reshape(
        DEN_ROWS * C)[:N].reshape(N, 1)
    den1 = parts[1, DEN_BASE:DEN_BASE + DEN_ROWS, :].reshape(
        DEN_ROWS * C)[:N].reshape(N, 1)

    return _combine(parts[0], parts[1], den0, den1, h, a_s, a_d, c,
                    bias.reshape(1, C))
